# baseline (speedup 1.0000x reference)
import functools

import jax
import jax.numpy as jnp
from jax import lax
from jax.experimental import pallas as pl
from jax.experimental.pallas import tpu as pltpu
from jax.experimental.pallas import tpu_sc as plsc

_BR = 256


def _vq_body(zz_ref, ee_ref, z_ref, e_ref,
             idx_ref, stats_ref,
             acc_ref, hist_ref, sq_ref):
    i = pl.program_id(0)
    nsteps = pl.num_programs(0)
    K = e_ref.shape[0]
    btot = nsteps * z_ref.shape[0]

    zb = z_ref[...]
    e = e_ref[...]
    zzb = zz_ref[...]
    ee = ee_ref[...]

    dots = lax.dot_general(zb, e, (((1,), (1,)), ((), ())),
                           preferred_element_type=jnp.float32)
    d = zzb + ee[None, :] + dots

    colf = lax.broadcasted_iota(jnp.int32, (1, K), 1).astype(jnp.float32)
    dmin = jnp.min(d, axis=1, keepdims=True)
    idxf = jnp.min(jnp.where(d == dmin, colf, jnp.float32(K)),
                   axis=1, keepdims=True)
    idx2 = idxf.astype(jnp.int32)
    onehot = (colf == idxf).astype(jnp.float32)

    p = jnp.exp(dmin - d)
    s = jnp.sum(p, axis=1, keepdims=True)
    soft_sum = jnp.sum(p / s, axis=0, keepdims=True)
    hist_c = jnp.sum(onehot, axis=0, keepdims=True)
    sq_c = jnp.sum(dmin)

    @pl.when(i == 0)
    def _init():
        acc_ref[...] = soft_sum
        hist_ref[...] = hist_c
        sq_ref[0] = sq_c

    @pl.when(i > 0)
    def _accum():
        acc_ref[...] += soft_sum
        hist_ref[...] += hist_c
        sq_ref[0] += sq_c

    idx_ref[...] = idx2[:, 0]

    @pl.when(i == nsteps - 1)
    def _finalize():
        avg = acc_ref[...] / btot
        ent = -jnp.sum(avg * jnp.log(avg + 1e-10))
        div = jnp.log(jnp.float32(K)) - ent
        probs = hist_ref[...] / btot
        perp = jnp.exp(-jnp.sum(probs * jnp.log(probs + 1e-10)))
        sqm = sq_ref[0] / (btot * z_ref.shape[1])
        stats_ref[0] = sqm * 0.25 + sqm + 0.1 * div
        stats_ref[1] = perp
        stats_ref[2] = div


def _vq_main(z, embedding, zz, ee, interpret=False):
    B, D = z.shape
    K = embedding.shape[0]
    grid = (B // _BR,)
    return pl.pallas_call(
        _vq_body,
        grid=grid,
        in_specs=[
            pl.BlockSpec((_BR, 1), lambda i: (i, 0)),
            pl.BlockSpec((K,), lambda i: (0,)),
            pl.BlockSpec((_BR, D), lambda i: (i, 0)),
            pl.BlockSpec((K, D), lambda i: (0, 0)),
        ],
        out_specs=[
            pl.BlockSpec((_BR,), lambda i: (i,)),
            pl.BlockSpec(memory_space=pltpu.SMEM),
        ],
        out_shape=[
            jax.ShapeDtypeStruct((B,), jnp.int32),
            jax.ShapeDtypeStruct((8,), jnp.float32),
        ],
        scratch_shapes=[
            pltpu.VMEM((1, K), jnp.float32),
            pltpu.VMEM((1, K), jnp.float32),
            pltpu.SMEM((1,), jnp.float32),
        ],
        interpret=interpret,
    )(zz, ee, z, embedding)


def _sc_gather(embedding, idx2d, B, D):
    info = plsc.get_sparse_core_info()
    NC, NS = info.num_cores, info.num_subcores
    NW = NC * NS
    nrows = idx2d.shape[0]
    rows_per_w = nrows // NW
    CHUNK = idx2d.shape[1]

    @functools.partial(
        pl.kernel,
        mesh=plsc.VectorSubcoreMesh(core_axis_name="c", subcore_axis_name="s"),
        out_type=jax.ShapeDtypeStruct((B, D), jnp.float32),
        scratch_types=[
            pltpu.VMEM((rows_per_w, CHUNK), jnp.int32),
            pltpu.VMEM((CHUNK, D), jnp.float32),
            pltpu.SemaphoreType.DMA,
        ],
    )
    def gather(table_hbm, idx_hbm, out_hbm, idx_v, rows_v, sem):
        wid = lax.axis_index("s") * NC + lax.axis_index("c")
        pltpu.sync_copy(idx_hbm.at[pl.ds(wid * rows_per_w, rows_per_w)], idx_v)
        for j in range(rows_per_w):
            pltpu.async_copy(table_hbm.at[idx_v.at[j]], rows_v, sem).wait()
            pltpu.sync_copy(
                rows_v,
                out_hbm.at[pl.ds((wid * rows_per_w + j) * CHUNK, CHUNK)])

    return gather(embedding, idx2d)


def kernel(z, embedding):
    B, D = z.shape
    K = embedding.shape[0]
    zz = jnp.sum(z * z, axis=1, keepdims=True)
    ee = jnp.sum(embedding * embedding, axis=1)
    indices, stats = _vq_main(z, embedding * -2.0, zz, ee)
    emb_pad = jnp.pad(embedding, ((0, 0), (0, 128 - D)))
    z_q = _sc_gather(emb_pad, indices.reshape(B // 128, 128), B, 128)[:, :D]
    return (z_q, stats[0], stats[1], indices, stats[2])


# speedup vs baseline: 1.0801x; 1.0801x over previous
import functools

import jax
import jax.numpy as jnp
from jax import lax
from jax.experimental import pallas as pl
from jax.experimental.pallas import tpu as pltpu
from jax.experimental.pallas import tpu_sc as plsc

_BR = 256


def _vq_body(zz_ref, ee_ref, z_ref, e_ref,
             idx_ref, stats_ref,
             acc_ref, hist_ref, sq_ref):
    i = pl.program_id(0)
    nsteps = pl.num_programs(0)
    K = e_ref.shape[0]
    btot = nsteps * z_ref.shape[0]

    zb = z_ref[...]
    e = e_ref[...]
    zzb = zz_ref[...]
    ee = ee_ref[...]

    dots = lax.dot_general(zb, e, (((1,), (1,)), ((), ())),
                           preferred_element_type=jnp.float32)
    d = zzb + ee[None, :] - 2.0 * dots

    colf = lax.broadcasted_iota(jnp.int32, (1, K), 1).astype(jnp.float32)
    dmin = jnp.min(d, axis=1, keepdims=True)
    idxf = jnp.min(jnp.where(d == dmin, colf, jnp.float32(K)),
                   axis=1, keepdims=True)
    idx2 = idxf.astype(jnp.int32)
    onehot = (colf == idxf).astype(jnp.float32)

    p = jnp.exp(dmin - d)
    s = jnp.sum(p, axis=1, keepdims=True)
    soft_sum = jnp.sum(p / s, axis=0, keepdims=True)
    hist_c = jnp.sum(onehot, axis=0, keepdims=True)
    sq_c = jnp.sum(dmin)

    @pl.when(i == 0)
    def _init():
        acc_ref[...] = soft_sum
        hist_ref[...] = hist_c
        sq_ref[0] = sq_c

    @pl.when(i > 0)
    def _accum():
        acc_ref[...] += soft_sum
        hist_ref[...] += hist_c
        sq_ref[0] += sq_c

    idx_ref[...] = idx2[:, 0]

    @pl.when(i == nsteps - 1)
    def _finalize():
        avg = acc_ref[...] / btot
        ent = -jnp.sum(avg * jnp.log(avg + 1e-10))
        div = jnp.log(jnp.float32(K)) - ent
        probs = hist_ref[...] / btot
        perp = jnp.exp(-jnp.sum(probs * jnp.log(probs + 1e-10)))
        sqm = sq_ref[0] / (btot * z_ref.shape[1])
        stats_ref[0] = sqm * 0.25 + sqm + 0.1 * div
        stats_ref[1] = perp
        stats_ref[2] = div


def _vq_main(z, embedding, zz, ee, interpret=False):
    B, D = z.shape
    K = embedding.shape[0]
    grid = (B // _BR,)
    return pl.pallas_call(
        _vq_body,
        grid=grid,
        in_specs=[
            pl.BlockSpec((_BR, 1), lambda i: (i, 0)),
            pl.BlockSpec((K,), lambda i: (0,)),
            pl.BlockSpec((_BR, D), lambda i: (i, 0)),
            pl.BlockSpec((K, D), lambda i: (0, 0)),
        ],
        out_specs=[
            pl.BlockSpec((_BR,), lambda i: (i,)),
            pl.BlockSpec(memory_space=pltpu.SMEM),
        ],
        out_shape=[
            jax.ShapeDtypeStruct((B,), jnp.int32),
            jax.ShapeDtypeStruct((8,), jnp.float32),
        ],
        scratch_shapes=[
            pltpu.VMEM((1, K), jnp.float32),
            pltpu.VMEM((1, K), jnp.float32),
            pltpu.SMEM((1,), jnp.float32),
        ],
        interpret=interpret,
    )(zz, ee, z, embedding)


def _sc_gather(embedding, idx2d, B, D):
    info = plsc.get_sparse_core_info()
    NC, NS = info.num_cores, info.num_subcores
    NW = NC * NS
    nrows = idx2d.shape[0]
    rows_per_w = nrows // NW
    CHUNK = idx2d.shape[1]

    @functools.partial(
        pl.kernel,
        mesh=plsc.VectorSubcoreMesh(core_axis_name="c", subcore_axis_name="s"),
        out_type=jax.ShapeDtypeStruct((B, D), jnp.float32),
        scratch_types=[
            pltpu.VMEM((rows_per_w, CHUNK), jnp.int32),
            pltpu.VMEM((CHUNK, D), jnp.float32),
            pltpu.SemaphoreType.DMA,
        ],
    )
    def gather(table_hbm, idx_hbm, out_hbm, idx_v, rows_v, sem):
        wid = lax.axis_index("s") * NC + lax.axis_index("c")
        pltpu.sync_copy(idx_hbm.at[pl.ds(wid * rows_per_w, rows_per_w)], idx_v)
        for j in range(rows_per_w):
            pltpu.async_copy(table_hbm.at[idx_v.at[j]], rows_v, sem).wait()
            pltpu.sync_copy(
                rows_v,
                out_hbm.at[pl.ds((wid * rows_per_w + j) * CHUNK, CHUNK)])

    return gather(embedding, idx2d)


def kernel(z, embedding):
    B, D = z.shape
    K = embedding.shape[0]
    zz = jnp.sum(z * z, axis=1, keepdims=True)
    ee = jnp.sum(embedding * embedding, axis=1)
    indices, stats = _vq_main(z, embedding, zz, ee)
    emb_pad = jnp.pad(embedding, ((0, 0), (0, 128 - D)))
    z_q = _sc_gather(emb_pad, indices.reshape(B // 128, 128), B, 128)[:, :D]
    return (z_q, stats[0], stats[1], indices, stats[2])


# speedup vs baseline: 1.1381x; 1.0537x over previous
import functools

import jax
import jax.numpy as jnp
from jax import lax
from jax.experimental import pallas as pl
from jax.experimental.pallas import tpu as pltpu
from jax.experimental.pallas import tpu_sc as plsc

_BR = 512


def _vq_body(zz_ref, ee_ref, z_ref, e_ref,
             idx_ref, stats_ref,
             acc_ref, hist_ref, sq_ref):
    i = pl.program_id(0)
    nsteps = pl.num_programs(0)
    K = e_ref.shape[0]
    btot = nsteps * z_ref.shape[0]

    zb = z_ref[...]
    e = e_ref[...]
    zzb = zz_ref[...]
    ee = ee_ref[...]

    dots = lax.dot_general(zb, e, (((1,), (1,)), ((), ())),
                           preferred_element_type=jnp.float32)
    d = zzb + ee[None, :] - 2.0 * dots

    colf = lax.broadcasted_iota(jnp.int32, (1, K), 1).astype(jnp.float32)
    dmin = jnp.min(d, axis=1, keepdims=True)
    idxf = jnp.min(jnp.where(d == dmin, colf, jnp.float32(K)),
                   axis=1, keepdims=True)
    idx2 = idxf.astype(jnp.int32)
    onehot = (colf == idxf).astype(jnp.float32)

    p = jnp.exp(dmin - d)
    s = jnp.sum(p, axis=1, keepdims=True)
    soft_sum = jnp.sum(p / s, axis=0, keepdims=True)
    hist_c = jnp.sum(onehot, axis=0, keepdims=True)
    sq_c = jnp.sum(dmin)

    @pl.when(i == 0)
    def _init():
        acc_ref[...] = soft_sum
        hist_ref[...] = hist_c
        sq_ref[0] = sq_c

    @pl.when(i > 0)
    def _accum():
        acc_ref[...] += soft_sum
        hist_ref[...] += hist_c
        sq_ref[0] += sq_c

    idx_ref[...] = idx2[:, 0]

    @pl.when(i == nsteps - 1)
    def _finalize():
        avg = acc_ref[...] / btot
        ent = -jnp.sum(avg * jnp.log(avg + 1e-10))
        div = jnp.log(jnp.float32(K)) - ent
        probs = hist_ref[...] / btot
        perp = jnp.exp(-jnp.sum(probs * jnp.log(probs + 1e-10)))
        sqm = sq_ref[0] / (btot * z_ref.shape[1])
        stats_ref[0] = sqm * 0.25 + sqm + 0.1 * div
        stats_ref[1] = perp
        stats_ref[2] = div


def _vq_main(z, embedding, zz, ee, interpret=False):
    B, D = z.shape
    K = embedding.shape[0]
    grid = (B // _BR,)
    return pl.pallas_call(
        _vq_body,
        grid=grid,
        in_specs=[
            pl.BlockSpec((_BR, 1), lambda i: (i, 0)),
            pl.BlockSpec((K,), lambda i: (0,)),
            pl.BlockSpec((_BR, D), lambda i: (i, 0)),
            pl.BlockSpec((K, D), lambda i: (0, 0)),
        ],
        out_specs=[
            pl.BlockSpec((_BR,), lambda i: (i,)),
            pl.BlockSpec(memory_space=pltpu.SMEM),
        ],
        out_shape=[
            jax.ShapeDtypeStruct((B,), jnp.int32),
            jax.ShapeDtypeStruct((8,), jnp.float32),
        ],
        scratch_shapes=[
            pltpu.VMEM((1, K), jnp.float32),
            pltpu.VMEM((1, K), jnp.float32),
            pltpu.SMEM((1,), jnp.float32),
        ],
        compiler_params=pltpu.CompilerParams(
            vmem_limit_bytes=110 * 1024 * 1024),
        interpret=interpret,
    )(zz, ee, z, embedding)


def _sc_gather(embedding, idx2d, B, D):
    info = plsc.get_sparse_core_info()
    NC, NS = info.num_cores, info.num_subcores
    NW = NC * NS
    nrows = idx2d.shape[0]
    rows_per_w = nrows // NW
    CHUNK = idx2d.shape[1]

    @functools.partial(
        pl.kernel,
        mesh=plsc.VectorSubcoreMesh(core_axis_name="c", subcore_axis_name="s"),
        out_type=jax.ShapeDtypeStruct((B, D), jnp.float32),
        scratch_types=[
            pltpu.VMEM((rows_per_w, CHUNK), jnp.int32),
            pltpu.VMEM((CHUNK, D), jnp.float32),
            pltpu.SemaphoreType.DMA,
        ],
    )
    def gather(table_hbm, idx_hbm, out_hbm, idx_v, rows_v, sem):
        wid = lax.axis_index("s") * NC + lax.axis_index("c")
        pltpu.sync_copy(idx_hbm.at[pl.ds(wid * rows_per_w, rows_per_w)], idx_v)
        for j in range(rows_per_w):
            pltpu.async_copy(table_hbm.at[idx_v.at[j]], rows_v, sem).wait()
            pltpu.sync_copy(
                rows_v,
                out_hbm.at[pl.ds((wid * rows_per_w + j) * CHUNK, CHUNK)])

    return gather(embedding, idx2d)


def kernel(z, embedding):
    B, D = z.shape
    K = embedding.shape[0]
    zz = jnp.sum(z * z, axis=1, keepdims=True)
    ee = jnp.sum(embedding * embedding, axis=1)
    indices, stats = _vq_main(z, embedding, zz, ee)
    emb_pad = jnp.pad(embedding, ((0, 0), (0, 128 - D)))
    z_q = _sc_gather(emb_pad, indices.reshape(B // 128, 128), B, 128)[:, :D]
    return (z_q, stats[0], stats[1], indices, stats[2])


# speedup vs baseline: 1.3145x; 1.1551x over previous
import functools

import jax
import jax.numpy as jnp
from jax import lax
from jax.experimental import pallas as pl
from jax.experimental.pallas import tpu as pltpu
from jax.experimental.pallas import tpu_sc as plsc

_BR = 512


def _vq_body(zz_ref, ee_ref, z_ref, e_ref,
             idx_ref, stats_ref,
             acc_ref, hist_ref, sq_ref):
    i = pl.program_id(0)
    nsteps = pl.num_programs(0)
    K = e_ref.shape[0]
    btot = nsteps * z_ref.shape[0]

    zb = z_ref[...]
    e = e_ref[...]
    zzb = zz_ref[...]
    ee = ee_ref[...]

    dots = lax.dot_general(zb, e, (((1,), (1,)), ((), ())),
                           preferred_element_type=jnp.float32)
    d = zzb + ee[None, :] - 2.0 * dots

    colf = lax.broadcasted_iota(jnp.int32, (1, K), 1).astype(jnp.float32)
    dmin = jnp.min(d, axis=1, keepdims=True)
    idxf = jnp.min(jnp.where(d == dmin, colf, jnp.float32(K)),
                   axis=1, keepdims=True)
    idx2 = idxf.astype(jnp.int32)
    onehot = (colf == idxf).astype(jnp.float32)

    p = jnp.exp(dmin - d)
    s1 = jnp.sum(p, axis=1)
    srow = (1.0 / s1)[None, :]
    soft_sum = lax.dot_general(srow, p, (((1,), (0,)), ((), ())),
                               preferred_element_type=jnp.float32)
    ones_row = jnp.ones((1, p.shape[0]), jnp.float32)
    hist_c = lax.dot_general(ones_row, onehot, (((1,), (0,)), ((), ())),
                             preferred_element_type=jnp.float32)
    sq_c = jnp.sum(dmin)

    @pl.when(i == 0)
    def _init():
        acc_ref[...] = soft_sum
        hist_ref[...] = hist_c
        sq_ref[0] = sq_c

    @pl.when(i > 0)
    def _accum():
        acc_ref[...] += soft_sum
        hist_ref[...] += hist_c
        sq_ref[0] += sq_c

    idx_ref[...] = idx2[:, 0]

    @pl.when(i == nsteps - 1)
    def _finalize():
        avg = acc_ref[...] / btot
        ent = -jnp.sum(avg * jnp.log(avg + 1e-10))
        div = jnp.log(jnp.float32(K)) - ent
        probs = hist_ref[...] / btot
        perp = jnp.exp(-jnp.sum(probs * jnp.log(probs + 1e-10)))
        sqm = sq_ref[0] / (btot * z_ref.shape[1])
        stats_ref[0] = sqm * 0.25 + sqm + 0.1 * div
        stats_ref[1] = perp
        stats_ref[2] = div


def _vq_main(z, embedding, zz, ee, interpret=False):
    B, D = z.shape
    K = embedding.shape[0]
    grid = (B // _BR,)
    return pl.pallas_call(
        _vq_body,
        grid=grid,
        in_specs=[
            pl.BlockSpec((_BR, 1), lambda i: (i, 0)),
            pl.BlockSpec((K,), lambda i: (0,)),
            pl.BlockSpec((_BR, D), lambda i: (i, 0)),
            pl.BlockSpec((K, D), lambda i: (0, 0)),
        ],
        out_specs=[
            pl.BlockSpec((_BR,), lambda i: (i,)),
            pl.BlockSpec(memory_space=pltpu.SMEM),
        ],
        out_shape=[
            jax.ShapeDtypeStruct((B,), jnp.int32),
            jax.ShapeDtypeStruct((8,), jnp.float32),
        ],
        scratch_shapes=[
            pltpu.VMEM((1, K), jnp.float32),
            pltpu.VMEM((1, K), jnp.float32),
            pltpu.SMEM((1,), jnp.float32),
        ],
        compiler_params=pltpu.CompilerParams(
            vmem_limit_bytes=110 * 1024 * 1024),
        interpret=interpret,
    )(zz, ee, z, embedding)


def _sc_gather(embedding, idx2d, B, D):
    info = plsc.get_sparse_core_info()
    NC, NS = info.num_cores, info.num_subcores
    NW = NC * NS
    nrows = idx2d.shape[0]
    rows_per_w = nrows // NW
    CHUNK = idx2d.shape[1]

    @functools.partial(
        pl.kernel,
        mesh=plsc.VectorSubcoreMesh(core_axis_name="c", subcore_axis_name="s"),
        out_type=jax.ShapeDtypeStruct((B, D), jnp.float32),
        scratch_types=[
            pltpu.VMEM((rows_per_w, CHUNK), jnp.int32),
            pltpu.VMEM((CHUNK, D), jnp.float32),
            pltpu.SemaphoreType.DMA,
        ],
    )
    def gather(table_hbm, idx_hbm, out_hbm, idx_v, rows_v, sem):
        wid = lax.axis_index("s") * NC + lax.axis_index("c")
        pltpu.sync_copy(idx_hbm.at[pl.ds(wid * rows_per_w, rows_per_w)], idx_v)
        for j in range(rows_per_w):
            pltpu.async_copy(table_hbm.at[idx_v.at[j]], rows_v, sem).wait()
            pltpu.sync_copy(
                rows_v,
                out_hbm.at[pl.ds((wid * rows_per_w + j) * CHUNK, CHUNK)])

    return gather(embedding, idx2d)


def kernel(z, embedding):
    B, D = z.shape
    K = embedding.shape[0]
    zz = jnp.sum(z * z, axis=1, keepdims=True)
    ee = jnp.sum(embedding * embedding, axis=1)
    indices, stats = _vq_main(z, embedding, zz, ee)
    emb_pad = jnp.pad(embedding, ((0, 0), (0, 128 - D)))
    z_q = _sc_gather(emb_pad, indices.reshape(B // 128, 128), B, 128)[:, :D]
    return (z_q, stats[0], stats[1], indices, stats[2])
